# Initial kernel scaffold; baseline (speedup 1.0000x reference)
import jax
import jax.numpy as jnp
from jax.experimental import pallas as pl


def kernel(x, edge_index, mask):
    raise NotImplementedError("write your pallas kernel here")


# speedup vs baseline: 5.8202x; 5.8202x over previous
import functools

import jax
import jax.numpy as jnp
from jax import lax
from jax.experimental import pallas as pl
from jax.experimental.pallas import tpu as pltpu
from jax.experimental.pallas import tpu_sc as plsc

N_NODES = 10000
N_EDGES = 320000
D_FEAT = 128
NUM_ITERS = 40

NC = 2
NS = 16
LANES = 16

NP = 10240
HALF = 64
CK = 128
CH = 157
EPT = CH * CK
ROWS_PT = NP // NS
NODE_CHUNKS = ROWS_PT // CK
GROUPS = HALF // LANES


def _fp_body(colx, rowx, dxm, cfx, xm, fcf, out_hbm, y_hbm,
             idxc_v, idxr_v, rows_v, acc_v, scale_v, bias_v, zero_v,
             acc_sh, sem):
    c = lax.axis_index("c")
    s = lax.axis_index("s")

    pltpu.sync_copy(colx.at[c, s], idxc_v)
    pltpu.sync_copy(rowx.at[s], idxr_v)

    def zrow(i, _):
        for g in range(GROUPS):
            zero_v[i, pl.ds(g * LANES, LANES)] = jnp.zeros((LANES,), jnp.float32)
        return 0
    lax.fori_loop(0, CK, zrow, 0)

    def init_chunk(k, _):
        base_h = s * ROWS_PT + k * CK
        base_g = c * NP + base_h
        pltpu.sync_copy(zero_v, acc_sh.at[pl.ds(base_h, CK)])
        pltpu.sync_copy(dxm.at[pl.ds(base_g, CK)], acc_v)
        pltpu.sync_copy(acc_v, y_hbm.at[pl.ds(base_g, CK)])
        return 0
    lax.fori_loop(0, NODE_CHUNKS, init_chunk, 0)
    plsc.subcore_barrier()

    def iteration(t, _):
        def edge_chunk(j, _):
            pltpu.async_copy(y_hbm.at[idxc_v.at[j]], rows_v, sem).wait()
            pltpu.sync_copy(rows_v, acc_sh.at[idxr_v.at[j]], add=True)
            return 0
        lax.fori_loop(0, CH, edge_chunk, 0)
        plsc.subcore_barrier()

        def node_chunk(k, _):
            base_h = s * ROWS_PT + k * CK
            base_g = c * NP + base_h
            pltpu.sync_copy(acc_sh.at[pl.ds(base_h, CK)], acc_v)

            @pl.when(t < NUM_ITERS - 1)
            def _load_mid():
                pltpu.sync_copy(cfx.at[pl.ds(base_g, CK)], scale_v)
                pltpu.sync_copy(dxm.at[pl.ds(base_g, CK)], bias_v)

            @pl.when(t == NUM_ITERS - 1)
            def _load_final():
                pltpu.sync_copy(fcf.at[pl.ds(base_g, CK)], scale_v)
                pltpu.sync_copy(xm.at[pl.ds(base_g, CK)], bias_v)

            def crow(i, _):
                for g in range(GROUPS):
                    sl = pl.ds(g * LANES, LANES)
                    acc_v[i, sl] = acc_v[i, sl] * scale_v[i, sl] + bias_v[i, sl]
                return 0
            lax.fori_loop(0, CK, crow, 0)

            @pl.when(t < NUM_ITERS - 1)
            def _store_mid():
                pltpu.sync_copy(acc_v, y_hbm.at[pl.ds(base_g, CK)])

            @pl.when(t == NUM_ITERS - 1)
            def _store_final():
                pltpu.sync_copy(acc_v, out_hbm.at[pl.ds(base_g, CK)])

            pltpu.sync_copy(zero_v, acc_sh.at[pl.ds(base_h, CK)])
            return 0
        lax.fori_loop(0, NODE_CHUNKS, node_chunk, 0)
        plsc.subcore_barrier()
        return 0

    lax.fori_loop(0, NUM_ITERS, iteration, 0)


@functools.lru_cache(maxsize=1)
def _build_kernel():
    mesh = plsc.VectorSubcoreMesh(core_axis_name="c", subcore_axis_name="s")
    return pl.kernel(
        _fp_body,
        out_type=(
            jax.ShapeDtypeStruct((2 * NP, HALF), jnp.float32),
            jax.ShapeDtypeStruct((2 * NP, HALF), jnp.float32),
        ),
        mesh=mesh,
        compiler_params=pltpu.CompilerParams(use_tc_tiling_on_sc=False),
        scratch_types=[
            pltpu.VMEM((CH, CK), jnp.int32),
            pltpu.VMEM((CH, CK), jnp.int32),
            pltpu.VMEM((CK, HALF), jnp.float32),
            pltpu.VMEM((CK, HALF), jnp.float32),
            pltpu.VMEM((CK, HALF), jnp.float32),
            pltpu.VMEM((CK, HALF), jnp.float32),
            pltpu.VMEM((CK, HALF), jnp.float32),
            pltpu.VMEM_SHARED((NP, HALF), jnp.float32),
            pltpu.SemaphoreType.DMA,
        ],
    )


def _split_pad(a):
    z = jnp.zeros((NP - N_NODES, HALF), jnp.float32)
    return jnp.concatenate([a[:, :HALF], z, a[:, HALF:], z], axis=0)


def kernel(x, edge_index, mask):
    row = edge_index[0]
    col = edge_index[1]

    deg = jax.ops.segment_sum(jnp.ones((N_EDGES,), jnp.float32), row,
                              num_segments=N_NODES)
    dis = jnp.where(deg > 0, lax.rsqrt(deg), 0.0)
    m2 = mask[:, None]
    dis2d = dis[:, None]
    x = x.astype(jnp.float32)
    dxm = _split_pad(jnp.where(m2, dis2d * x, 0.0))
    cfx = _split_pad(jnp.broadcast_to(
        jnp.where(mask, 0.0, dis * dis)[:, None], (N_NODES, D_FEAT)))
    xm = _split_pad(jnp.where(m2, x, 0.0))
    fcf = _split_pad(jnp.broadcast_to(
        jnp.where(mask, 0.0, dis)[:, None], (N_NODES, D_FEAT)))

    pad = NS * EPT - N_EDGES
    colp = jnp.concatenate([col, jnp.full((pad,), N_NODES, jnp.int32)])
    rowp = jnp.concatenate([row, jnp.full((pad,), N_NODES, jnp.int32)])
    colr = colp.reshape(NS, CH, CK)
    colx = jnp.stack([colr, colr + NP])
    rowx = rowp.reshape(NS, CH, CK)

    out2, _ = _build_kernel()(colx, rowx, dxm, cfx, xm, fcf)
    return jnp.concatenate([out2[:N_NODES], out2[NP:NP + N_NODES]], axis=1)
